# baseline (speedup 1.0000x reference)
import functools

import jax
import jax.numpy as jnp
from jax import lax
from jax.experimental import pallas as pl
from jax.experimental.pallas import tpu as pltpu
from jax.experimental.pallas import tpu_sc as plsc

_B = 4
_L = 4
_P = 128
_M = 192 * 256
_NC = 2
_NS = 16
_NW = _NC * _NS
_CPB = _NW // _B
_MW = _M // _CPB
_NV = _MW // 16
_NBKT = 136
_DUMMY = 129
_BKTN = _NBKT * 16
_REG = _L * _BKTN
_BIG = 1e10


def _sort16(x):
    return lax.sort(x, dimension=0)


def _bitonic(vs):
    if len(vs) == 1:
        return [_sort16(vs[0])]
    h = len(vs) // 2
    los = [jnp.minimum(a, b) for a, b in zip(vs[:h], vs[h:])]
    his = [jnp.maximum(a, b) for a, b in zip(vs[:h], vs[h:])]
    return _bitonic(los) + _bitonic(his)


def _merge(a, b):
    rb = [lax.rev(x, (0,)) for x in reversed(b)]
    los = [jnp.minimum(x, y) for x, y in zip(a, rb)]
    his = [jnp.maximum(x, y) for x, y in zip(a, rb)]
    return _bitonic(los) + _bitonic(his)


def _sort_vregs(vs):
    blocks = [[_sort16(v)] for v in vs]
    while len(blocks) > 1:
        blocks = [_merge(blocks[i], blocks[i + 1])
                  for i in range(0, len(blocks), 2)]
    return blocks[0]


def _sc_body(y_hbm, eb_hbm, omax_hbm, omin_hbm, otail_hbm, ocs_hbm, yv, ev,
             cv, s0, s1, s2, s3,
             mxe0, mxe1, mxe2, mxe3, mxo0, mxo1, mxo2, mxo3,
             mne0, mne1, mne2, mne3, mno0, mno1, mno2, mno3, tail):
    wid = lax.axis_index("s") * _NC + lax.axis_index("c")
    b = wid // _CPB
    chunk = wid % _CPB
    rows = 192 // _CPB
    pltpu.sync_copy(y_hbm.at[b, pl.ds(chunk * rows, rows)], yv)
    pltpu.sync_copy(eb_hbm.at[b], ev)

    srefs = (s0, s1, s2, s3)
    mxrefs = ((mxe0, mxe1, mxe2, mxe3), (mxo0, mxo1, mxo2, mxo3))
    mnrefs = ((mne0, mne1, mne2, mne3), (mno0, mno1, mno2, mno3))
    lane = lax.iota(jnp.int32, 16)
    neg1 = jnp.full((16,), -1.0, jnp.float32)
    two = jnp.full((16,), 2.0, jnp.float32)

    for l in range(_L):
        cvs = [0.5 * (ev[l, pl.ds(k * 16, 16)] + ev[l, pl.ds(k * 16 + 1, 16)])
               for k in range(_P // 16)]
        svs = _sort_vregs(cvs)
        for k, v in enumerate(svs):
            cv[pl.ds(l * _P + k * 16, 16)] = v

    @pl.when(chunk == 0)
    def _write_csort():
        pltpu.sync_copy(cv, ocs_hbm.at[b])

    def init_step(i, carry):
        off = pl.multiple_of(i * 16, 16)
        for par in range(2):
            for l in range(_L):
                mxrefs[par][l][pl.ds(off, 16)] = neg1
                mnrefs[par][l][pl.ds(off, 16)] = two
        return carry
    lax.fori_loop(0, _NBKT, init_step, 0)

    zero = jnp.zeros((16,), jnp.float32)

    topv = []
    for l in range(_L):
        lb = l * _P
        tv = {}
        for ti in (127, 63, 31, 95, 15, 47, 79, 111):
            tv[ti] = plsc.load_gather(
                cv, [jnp.full((16,), lb + ti, jnp.int32)])
        topv.append(tv)

    chy_res = []
    cnt_res = None
    for lev in range(_L):
        def _loop1(g, carry, lev=lev):
            lbase = lev * _P
            tv = topv[lev]
            off = pl.multiple_of(g * 16, 16)
            y16 = yv[g // 16, pl.ds(pl.multiple_of((g % 16) * 16, 16), 16)]
            valid = y16 >= 0.001
            pos = jnp.where(tv[127] <= y16, 128, 0).astype(jnp.int32)
            pos = jnp.where((pos == 0) & (tv[63] <= y16), 64, pos)
            pv = jnp.where(pos == 64, tv[95], tv[31])
            pos = jnp.where((pos <= 64) & (pv <= y16), pos + 32, pos)
            pv = jnp.where(pos >= 64,
                           jnp.where(pos >= 96, tv[111], tv[79]),
                           jnp.where(pos >= 32, tv[47], tv[15]))
            pos = jnp.where((pos <= 96) & (pv <= y16), pos + 16, pos)
            for bit in (8, 4, 2, 1):
                nxt = pos + bit
                cidx = jnp.minimum(nxt - 1, _P - 1)
                cval = plsc.load_gather(cv, [lbase + cidx])
                pos = jnp.where((nxt <= _P) & (cval <= y16), nxt, pos)
            clo = plsc.load_gather(cv, [lbase + jnp.maximum(pos - 1, 0)])
            chi = plsc.load_gather(cv, [lbase + jnp.minimum(pos, _P - 1)])
            dl = y16 - clo
            dh = chi - y16
            dlo = jnp.where(pos > 0, dl * dl, _BIG)
            dhi = jnp.where(pos < _P, dh * dh, _BIG)
            dmin = jnp.minimum(dlo, dhi)
            chy = carry[0] + jnp.where(valid, dmin, 0.0)
            sbkt = jnp.where(valid, pos, _DUMMY)
            srefs[lev][pl.ds(off, 16)] = sbkt * 16 + lane
            if lev == 0:
                cnt = carry[1] + jnp.where(valid, 1.0, 0.0).astype(jnp.float32)
                return (chy, cnt)
            return (chy,)
        carry0 = (zero, zero) if lev == 0 else (zero,)
        out = plsc.parallel_loop(0, _NV, unroll=2, carry=carry0)(_loop1)
        chy_res.append(out[0])
        if lev == 0:
            cnt_res = out[1]
    res = tuple(chy_res) + (cnt_res,)

    def step2(g, carry):
        updates = []
        for par in range(2):
            g2 = 2 * g + par
            off = pl.multiple_of(g2 * 16, 16)
            y16 = yv[g2 // 16, pl.ds(pl.multiple_of((g2 % 16) * 16, 16), 16)]
            for l in range(_L):
                idx = srefs[l][pl.ds(off, 16)]
                curmax = plsc.load_gather(mxrefs[par][l], [idx])
                curmin = plsc.load_gather(mnrefs[par][l], [idx])
                updates.append((par, l, idx, jnp.maximum(curmax, y16),
                                jnp.minimum(curmin, y16)))
        for par, l, idx, newmax, newmin in updates:
            plsc.store_scatter(mxrefs[par][l], [idx], newmax)
            plsc.store_scatter(mnrefs[par][l], [idx], newmin)
        return carry
    lax.fori_loop(0, _NV // 2, step2, 0)

    def merge_step(i, carry):
        off = pl.multiple_of(i * 16, 16)
        for l in range(_L):
            mxrefs[0][l][pl.ds(off, 16)] = jnp.maximum(
                mxrefs[0][l][pl.ds(off, 16)], mxrefs[1][l][pl.ds(off, 16)])
            mnrefs[0][l][pl.ds(off, 16)] = jnp.minimum(
                mnrefs[0][l][pl.ds(off, 16)], mnrefs[1][l][pl.ds(off, 16)])
        return carry
    lax.fori_loop(0, _NBKT, merge_step, 0)

    for l in range(_L):
        tail[pl.ds(l * 16, 16)] = res[l]
    tail[pl.ds(_L * 16, 16)] = res[_L]
    for l in range(_L):
        pltpu.sync_copy(mxrefs[0][l], omax_hbm.at[wid, pl.ds(l * _BKTN, _BKTN)])
        pltpu.sync_copy(mnrefs[0][l], omin_hbm.at[wid, pl.ds(l * _BKTN, _BKTN)])
    pltpu.sync_copy(tail, otail_hbm.at[wid])


_sc_call = functools.partial(
    pl.kernel,
    out_type=(
        jax.ShapeDtypeStruct((_NW, _REG), jnp.float32),
        jax.ShapeDtypeStruct((_NW, _REG), jnp.float32),
        jax.ShapeDtypeStruct((_NW, (_L + 1) * 16), jnp.float32),
        jax.ShapeDtypeStruct((_B, _L * _P), jnp.float32),
    ),
    mesh=plsc.VectorSubcoreMesh(
        core_axis_name="c", subcore_axis_name="s",
        num_cores=_NC, num_subcores=_NS),
    scratch_types=[
        pltpu.VMEM((192 // _CPB, 256), jnp.float32),
        pltpu.VMEM((_L, _P + 1), jnp.float32),
        pltpu.VMEM((_L * _P,), jnp.float32),
    ] + [pltpu.VMEM((_MW,), jnp.int32)] * _L
      + [pltpu.VMEM((_BKTN,), jnp.float32)] * (4 * _L)
      + [pltpu.VMEM(((_L + 1) * 16,), jnp.float32)],
    compiler_params=pltpu.CompilerParams(
        needs_layout_passes=False, use_tc_tiling_on_sc=False),
)(_sc_body)


def _ep_body(maxb_ref, minb_ref, cnt_ref, cs_ref, out_ref):
    mx = maxb_ref[...]
    mn = minb_ref[...]
    mxr = jnp.concatenate(
        [jnp.max(mx[b * _CPB:(b + 1) * _CPB], axis=0, keepdims=True)
         for b in range(_B)], axis=0)
    mnr = jnp.concatenate(
        [jnp.min(mn[b * _CPB:(b + 1) * _CPB], axis=0, keepdims=True)
         for b in range(_B)], axis=0)
    for k in (1, 2, 4, 8):
        mxr = jnp.maximum(mxr, pltpu.roll(mxr, _REG - k, axis=1))
        mnr = jnp.minimum(mnr, pltpu.roll(mnr, _REG - k, axis=1))
    col = lax.broadcasted_iota(jnp.int32, (_P, _BKTN), 1)
    p_i = lax.broadcasted_iota(jnp.int32, (_P, _BKTN), 0)
    s_i = col // 16
    is_b = (col % 16) == 0
    predm = is_b & (s_i <= p_i)
    succm = is_b & (s_i > p_i) & (s_i <= _P)
    c = cs_ref[...]
    t = cnt_ref[...]
    tr = jnp.concatenate(
        [jnp.sum(t[b * _CPB:(b + 1) * _CPB], axis=0, keepdims=True)
         for b in range(_B)], axis=0)
    lanes80 = lax.broadcasted_iota(jnp.int32, (_B, 80), 1)
    lengths = jnp.sum(jnp.where(lanes80 >= _L * 16, tr, 0.0), axis=1)
    loss = jnp.float32(0.0)
    for l in range(_L):
        slab_x = mxr[:, l * _BKTN:(l + 1) * _BKTN]
        slab_n = mnr[:, l * _BKTN:(l + 1) * _BKTN]
        pred = jnp.max(jnp.where(predm[None], slab_x[:, None, :], -1.0),
                       axis=2)
        succ = jnp.min(jnp.where(succm[None], slab_n[:, None, :], 2.0),
                       axis=2)
        cl = c[:, l, :]
        dx = jnp.minimum(
            jnp.where(pred > -0.5, (cl - pred) ** 2, _BIG),
            jnp.where(succ < 1.5, (succ - cl) ** 2, _BIG))
        chamx = jnp.mean(dx, axis=1)
        chy_l = jnp.sum(
            jnp.where((lanes80 >= l * 16) & (lanes80 < (l + 1) * 16), tr, 0.0),
            axis=1)
        loss = loss + jnp.sum(chamx + chy_l / lengths)
    out_ref[...] = (loss / jnp.float32(_B))[None, None]


def kernel(bins, target_depth_maps):
    edges = bins.transpose(1, 0, 2)
    omax, omin, otail, ocs = _sc_call(target_depth_maps, edges)
    out = pl.pallas_call(
        _ep_body,
        out_shape=jax.ShapeDtypeStruct((1, 1), jnp.float32),
    )(omax, omin, otail, ocs.reshape(_B, _L, _P))
    return out[0, 0]


# speedup vs baseline: 1.0009x; 1.0009x over previous
import functools

import jax
import jax.numpy as jnp
from jax import lax
from jax.experimental import pallas as pl
from jax.experimental.pallas import tpu as pltpu
from jax.experimental.pallas import tpu_sc as plsc

_B = 4
_L = 4
_P = 128
_M = 192 * 256
_NC = 2
_NS = 16
_NW = _NC * _NS
_CPB = _NW // _B
_MW = _M // _CPB
_NV = _MW // 16
_NBKT = 136
_DUMMY = 129
_BKTN = _NBKT * 16
_REG = _L * _BKTN
_BIG = 1e10


def _sort16(x):
    return lax.sort(x, dimension=0)


def _bitonic(vs):
    if len(vs) == 1:
        return [_sort16(vs[0])]
    h = len(vs) // 2
    los = [jnp.minimum(a, b) for a, b in zip(vs[:h], vs[h:])]
    his = [jnp.maximum(a, b) for a, b in zip(vs[:h], vs[h:])]
    return _bitonic(los) + _bitonic(his)


def _merge(a, b):
    rb = [lax.rev(x, (0,)) for x in reversed(b)]
    los = [jnp.minimum(x, y) for x, y in zip(a, rb)]
    his = [jnp.maximum(x, y) for x, y in zip(a, rb)]
    return _bitonic(los) + _bitonic(his)


def _sort_vregs(vs):
    blocks = [[_sort16(v)] for v in vs]
    while len(blocks) > 1:
        blocks = [_merge(blocks[i], blocks[i + 1])
                  for i in range(0, len(blocks), 2)]
    return blocks[0]


def _sc_body(y_hbm, eb_hbm, omax_hbm, omin_hbm, otail_hbm, ocs_hbm, yv, ev,
             cv, s0, s1, s2, s3,
             mxe0, mxe1, mxe2, mxe3, mxo0, mxo1, mxo2, mxo3,
             mne0, mne1, mne2, mne3, mno0, mno1, mno2, mno3, tail):
    wid = lax.axis_index("s") * _NC + lax.axis_index("c")
    b = wid // _CPB
    chunk = wid % _CPB
    rows = 192 // _CPB
    pltpu.sync_copy(y_hbm.at[b, pl.ds(chunk * rows, rows)], yv)
    pltpu.sync_copy(eb_hbm.at[b], ev)

    srefs = (s0, s1, s2, s3)
    mxrefs = ((mxe0, mxe1, mxe2, mxe3), (mxo0, mxo1, mxo2, mxo3))
    mnrefs = ((mne0, mne1, mne2, mne3), (mno0, mno1, mno2, mno3))
    lane = lax.iota(jnp.int32, 16)
    neg1 = jnp.full((16,), -1.0, jnp.float32)
    two = jnp.full((16,), 2.0, jnp.float32)

    for l in range(_L):
        cvs = [0.5 * (ev[l, pl.ds(k * 16, 16)] + ev[l, pl.ds(k * 16 + 1, 16)])
               for k in range(_P // 16)]
        svs = _sort_vregs(cvs)
        for k, v in enumerate(svs):
            cv[pl.ds(l * _P + k * 16, 16)] = v

    @pl.when(chunk == 0)
    def _write_csort():
        pltpu.sync_copy(cv, ocs_hbm.at[b])

    def init_step(i, carry):
        off = pl.multiple_of(i * 16, 16)
        for par in range(2):
            for l in range(_L):
                mxrefs[par][l][pl.ds(off, 16)] = neg1
                mnrefs[par][l][pl.ds(off, 16)] = two
        return carry
    lax.fori_loop(0, _NBKT, init_step, 0)

    zero = jnp.zeros((16,), jnp.float32)

    topv = []
    for l in range(_L):
        lb = l * _P
        tv = {}
        for ti in (127, 63, 31, 95, 15, 47, 79, 111):
            tv[ti] = plsc.load_gather(
                cv, [jnp.full((16,), lb + ti, jnp.int32)])
        topv.append(tv)

    chy_res = []
    cnt_res = None
    for lev in range(_L):
        def _loop1(g, carry, lev=lev):
            lbase = lev * _P
            tv = topv[lev]
            off = pl.multiple_of(g * 16, 16)
            y16 = yv[g // 16, pl.ds(pl.multiple_of((g % 16) * 16, 16), 16)]
            valid = y16 >= 0.001
            pos = jnp.where(tv[127] <= y16, 128, 0).astype(jnp.int32)
            pos = jnp.where((pos == 0) & (tv[63] <= y16), 64, pos)
            pv = jnp.where(pos == 64, tv[95], tv[31])
            pos = jnp.where((pos <= 64) & (pv <= y16), pos + 32, pos)
            pv = jnp.where(pos >= 64,
                           jnp.where(pos >= 96, tv[111], tv[79]),
                           jnp.where(pos >= 32, tv[47], tv[15]))
            pos = jnp.where((pos <= 96) & (pv <= y16), pos + 16, pos)
            for bit in (8, 4, 2, 1):
                nxt = pos + bit
                cidx = jnp.minimum(nxt - 1, _P - 1)
                cval = plsc.load_gather(cv, [lbase + cidx])
                pos = jnp.where((nxt <= _P) & (cval <= y16), nxt, pos)
            clo = plsc.load_gather(cv, [lbase + jnp.maximum(pos - 1, 0)])
            chi = plsc.load_gather(cv, [lbase + jnp.minimum(pos, _P - 1)])
            dl = y16 - clo
            dh = chi - y16
            dlo = jnp.where(pos > 0, dl * dl, _BIG)
            dhi = jnp.where(pos < _P, dh * dh, _BIG)
            dmin = jnp.minimum(dlo, dhi)
            chy = carry[0] + jnp.where(valid, dmin, 0.0)
            sbkt = jnp.where(valid, pos, _DUMMY)
            srefs[lev][pl.ds(off, 16)] = sbkt * 16 + lane
            if lev == 0:
                cnt = carry[1] + jnp.where(valid, 1.0, 0.0).astype(jnp.float32)
                return (chy, cnt)
            return (chy,)
        carry0 = (zero, zero) if lev == 0 else (zero,)
        out = plsc.parallel_loop(0, _NV, unroll=2, carry=carry0)(_loop1)
        chy_res.append(out[0])
        if lev == 0:
            cnt_res = out[1]
    res = tuple(chy_res) + (cnt_res,)

    def step2(g, carry):
        updates = []
        for par in range(2):
            g2 = 2 * g + par
            off = pl.multiple_of(g2 * 16, 16)
            y16 = yv[g2 // 16, pl.ds(pl.multiple_of((g2 % 16) * 16, 16), 16)]
            for l in range(_L):
                idx = srefs[l][pl.ds(off, 16)]
                curmax = plsc.load_gather(mxrefs[par][l], [idx])
                curmin = plsc.load_gather(mnrefs[par][l], [idx])
                updates.append((par, l, idx, jnp.maximum(curmax, y16),
                                jnp.minimum(curmin, y16)))
        for par, l, idx, newmax, newmin in updates:
            plsc.store_scatter(mxrefs[par][l], [idx], newmax)
            plsc.store_scatter(mnrefs[par][l], [idx], newmin)
        return carry
    lax.fori_loop(0, _NV // 2, step2, 0)

    def merge_step(i, carry):
        off = pl.multiple_of(i * 16, 16)
        for l in range(_L):
            mxrefs[0][l][pl.ds(off, 16)] = jnp.maximum(
                mxrefs[0][l][pl.ds(off, 16)], mxrefs[1][l][pl.ds(off, 16)])
            mnrefs[0][l][pl.ds(off, 16)] = jnp.minimum(
                mnrefs[0][l][pl.ds(off, 16)], mnrefs[1][l][pl.ds(off, 16)])
        return carry
    lax.fori_loop(0, _NBKT, merge_step, 0)

    for l in range(_L):
        tail[pl.ds(l * 16, 16)] = res[l]
    tail[pl.ds(_L * 16, 16)] = res[_L]
    for l in range(_L):
        pltpu.sync_copy(mxrefs[0][l], omax_hbm.at[wid, pl.ds(l * _BKTN, _BKTN)])
        pltpu.sync_copy(mnrefs[0][l], omin_hbm.at[wid, pl.ds(l * _BKTN, _BKTN)])
    pltpu.sync_copy(tail, otail_hbm.at[wid])


@functools.lru_cache(maxsize=1)
def _sc_call_cached():
    return functools.partial(
        pl.kernel,
        out_type=(
            jax.ShapeDtypeStruct((_NW, _REG), jnp.float32),
            jax.ShapeDtypeStruct((_NW, _REG), jnp.float32),
            jax.ShapeDtypeStruct((_NW, (_L + 1) * 16), jnp.float32),
            jax.ShapeDtypeStruct((_B, _L * _P), jnp.float32),
        ),
        mesh=plsc.VectorSubcoreMesh(
            core_axis_name="c", subcore_axis_name="s",
            num_cores=_NC, num_subcores=_NS),
        scratch_types=[
            pltpu.VMEM((192 // _CPB, 256), jnp.float32),
            pltpu.VMEM((_L, _P + 1), jnp.float32),
            pltpu.VMEM((_L * _P,), jnp.float32),
        ] + [pltpu.VMEM((_MW,), jnp.int32)] * _L
          + [pltpu.VMEM((_BKTN,), jnp.float32)] * (4 * _L)
          + [pltpu.VMEM(((_L + 1) * 16,), jnp.float32)],
        compiler_params=pltpu.CompilerParams(
            needs_layout_passes=False, use_tc_tiling_on_sc=False),
    )(_sc_body)


def _ep_body(maxb_ref, minb_ref, cnt_ref, cs_ref, out_ref):
    mx = maxb_ref[...]
    mn = minb_ref[...]
    mxr = jnp.concatenate(
        [jnp.max(mx[b * _CPB:(b + 1) * _CPB], axis=0, keepdims=True)
         for b in range(_B)], axis=0)
    mnr = jnp.concatenate(
        [jnp.min(mn[b * _CPB:(b + 1) * _CPB], axis=0, keepdims=True)
         for b in range(_B)], axis=0)
    for k in (1, 2, 4, 8):
        mxr = jnp.maximum(mxr, pltpu.roll(mxr, _REG - k, axis=1))
        mnr = jnp.minimum(mnr, pltpu.roll(mnr, _REG - k, axis=1))
    col = lax.broadcasted_iota(jnp.int32, (_P, _BKTN), 1)
    p_i = lax.broadcasted_iota(jnp.int32, (_P, _BKTN), 0)
    s_i = col // 16
    is_b = (col % 16) == 0
    predm = is_b & (s_i <= p_i)
    succm = is_b & (s_i > p_i) & (s_i <= _P)
    c = cs_ref[...]
    t = cnt_ref[...]
    tr = jnp.concatenate(
        [jnp.sum(t[b * _CPB:(b + 1) * _CPB], axis=0, keepdims=True)
         for b in range(_B)], axis=0)
    lanes80 = lax.broadcasted_iota(jnp.int32, (_B, 80), 1)
    lengths = jnp.sum(jnp.where(lanes80 >= _L * 16, tr, 0.0), axis=1)
    loss = jnp.float32(0.0)
    for l in range(_L):
        slab_x = mxr[:, l * _BKTN:(l + 1) * _BKTN]
        slab_n = mnr[:, l * _BKTN:(l + 1) * _BKTN]
        pred = jnp.max(jnp.where(predm[None], slab_x[:, None, :], -1.0),
                       axis=2)
        succ = jnp.min(jnp.where(succm[None], slab_n[:, None, :], 2.0),
                       axis=2)
        cl = c[:, l, :]
        dx = jnp.minimum(
            jnp.where(pred > -0.5, (cl - pred) ** 2, _BIG),
            jnp.where(succ < 1.5, (succ - cl) ** 2, _BIG))
        chamx = jnp.mean(dx, axis=1)
        chy_l = jnp.sum(
            jnp.where((lanes80 >= l * 16) & (lanes80 < (l + 1) * 16), tr, 0.0),
            axis=1)
        loss = loss + jnp.sum(chamx + chy_l / lengths)
    out_ref[...] = (loss / jnp.float32(_B))[None, None]


def kernel(bins, target_depth_maps):
    edges = bins.transpose(1, 0, 2)
    omax, omin, otail, ocs = _sc_call_cached()(target_depth_maps, edges)
    out = pl.pallas_call(
        _ep_body,
        out_shape=jax.ShapeDtypeStruct((1, 1), jnp.float32),
    )(omax, omin, otail, ocs.reshape(_B, _L, _P))
    return out[0, 0]
